# baseline (speedup 1.0000x reference)
import functools

import jax
import jax.numpy as jnp
from jax import lax
from jax.experimental import pallas as pl
from jax.experimental.pallas import tpu as pltpu
from jax.experimental.pallas import tpu_sc as plsc

NSIZE = 1000000
EM = 64
B = 4096
NNEG = 20

NW = 32
R = B * (2 + 2 * NNEG)
PER_W = R // NW
CHUNK = 768
NCH = PER_W // CHUNK


def _gather_rows_sc(table, idx_all):
    mesh = plsc.VectorSubcoreMesh(core_axis_name="c", subcore_axis_name="s")

    @functools.partial(
        pl.kernel,
        out_type=jax.ShapeDtypeStruct((R, EM), jnp.float32),
        mesh=mesh,
        compiler_params=pltpu.CompilerParams(use_tc_tiling_on_sc=False),
        scratch_types=[
            pltpu.VMEM((PER_W,), jnp.int32),
            pltpu.VMEM((CHUNK, EM), jnp.float32),
            pltpu.SemaphoreType.DMA,
        ],
    )
    def gather_kernel(table_hbm, idx_hbm, out_hbm, idx_v, rows_v, sem):
        wid = lax.axis_index("s") * 2 + lax.axis_index("c")
        base = wid * PER_W
        pltpu.sync_copy(idx_hbm.at[pl.ds(base, PER_W)], idx_v)
        for c in range(NCH):
            pltpu.async_copy(
                table_hbm.at[idx_v.at[pl.ds(c * CHUNK, CHUNK)]], rows_v, sem
            ).wait()
            pltpu.sync_copy(rows_v, out_hbm.at[pl.ds(base + c * CHUNK, CHUNK)])

    return gather_kernel(table, idx_all)


BE = 512
GRID = B // BE


def _score_tc_body(e1, e2, n1, n2, kd, wab, prm, inten_o, surv_o):
    w0a = wab[:, 0]
    w1a = wab[:, 1]
    w0b = wab[:, 2]
    w1b = wab[:, 3]
    b0 = prm[0]
    b1 = prm[1]
    psi0 = prm[2]
    psi1 = prm[3]

    e1v = e1[...]
    e2v = e2[...]
    n1v = n1[...]
    n2v = n2[...]

    a0_e1 = jnp.sum(e1v * w0a, axis=-1)
    a1_e1 = jnp.sum(e1v * w1a, axis=-1)
    b0_e2 = jnp.sum(e2v * w0b, axis=-1)
    b1_e2 = jnp.sum(e2v * w1b, axis=-1)

    a0_n1 = jnp.sum(n1v * w0a, axis=-1)
    a1_n1 = jnp.sum(n1v * w1a, axis=-1)
    b0_n2 = jnp.sum(n2v * w0b, axis=-1)
    b1_n2 = jnp.sum(n2v * w1b, axis=-1)

    sc0 = a0_e1 + b0_e2 + b0
    sc1 = a1_e1 + b1_e2 + b1
    k0 = kd[0, :] == 0
    sck = jnp.where(k0, sc0, sc1)
    psik = jnp.where(k0, psi0, psi1)
    inten_o[0, :] = psik * jnp.log1p(jnp.exp(sck / psik))

    s10 = a0_e1[:, None] + b0_n2 + b0
    s11 = a1_e1[:, None] + b1_n2 + b1
    s20 = a0_n1 + b0_e2[:, None] + b0
    s21 = a1_n1 + b1_e2[:, None] + b1
    sp = lambda s, p: p * jnp.log1p(jnp.exp(s / p))
    acc = sp(s10, psi0) + sp(s11, psi1) + sp(s20, psi0) + sp(s21, psi1)
    surv_o[0, :] = jnp.sum(acc, axis=-1) * (1.0 / NNEG)


def _score_tc(e1, e2, n1, n2, kd, wab, prm):
    out_shapes = (
        jax.ShapeDtypeStruct((1, B), jnp.float32),
        jax.ShapeDtypeStruct((1, B), jnp.float32),
    )
    return pl.pallas_call(
        _score_tc_body,
        grid=(GRID,),
        in_specs=[
            pl.BlockSpec((BE, EM), lambda i: (i, 0)),
            pl.BlockSpec((BE, EM), lambda i: (i, 0)),
            pl.BlockSpec((BE, NNEG, EM), lambda i: (i, 0, 0)),
            pl.BlockSpec((BE, NNEG, EM), lambda i: (i, 0, 0)),
            pl.BlockSpec((1, BE), lambda i: (0, i)),
            pl.BlockSpec((EM, 4), lambda i: (0, 0)),
            pl.BlockSpec(memory_space=pltpu.SMEM),
        ],
        out_specs=(
            pl.BlockSpec((1, BE), lambda i: (0, i)),
            pl.BlockSpec((1, BE), lambda i: (0, i)),
        ),
        out_shape=out_shapes,
    )(e1, e2, n1, n2, kd, wab, prm)


def kernel(table, W0, b0, W1, b1, psi, events, negs):
    v1 = events[:, 0].astype(jnp.int32)
    v2 = events[:, 1].astype(jnp.int32)
    kd = events[:, 4].astype(jnp.int32)[None, :]
    n1i = negs[:, :, 0].astype(jnp.int32).reshape(-1)
    n2i = negs[:, :, 1].astype(jnp.int32).reshape(-1)

    idx_all = jnp.concatenate([v1, v2, n1i, n2i])
    g = _gather_rows_sc(table, idx_all)

    e1 = g[:B]
    e2 = g[B:2 * B]
    n1 = g[2 * B:2 * B + B * NNEG].reshape(B, NNEG, EM)
    n2 = g[2 * B + B * NNEG:].reshape(B, NNEG, EM)

    wab = jnp.concatenate([W0[:EM], W1[:EM], W0[EM:], W1[EM:]], axis=1)
    prm = jnp.stack([b0[0], b1[0], psi[0, 0], psi[1, 0]])

    inten, surv = _score_tc(e1, e2, n1, n2, kd, wab, prm)
    return inten, surv


# speedup vs baseline: 1.2572x; 1.2572x over previous
import functools

import jax
import jax.numpy as jnp
from jax import lax
from jax.experimental import pallas as pl
from jax.experimental.pallas import tpu as pltpu
from jax.experimental.pallas import tpu_sc as plsc

NSIZE = 1000000
EM = 64
B = 4096
NNEG = 20

NW = 32
R = B * (2 + 2 * NNEG)
PER_W = R // NW
CHUNK = 768
NCH = PER_W // CHUNK

BE = 512
GRID = B // BE
NEG_BLKS = B * NNEG // BE


def _gather_rows_sc(table, idx_all):
    mesh = plsc.VectorSubcoreMesh(core_axis_name="c", subcore_axis_name="s")

    @functools.partial(
        pl.kernel,
        out_type=jax.ShapeDtypeStruct((R, EM), jnp.float32),
        mesh=mesh,
        compiler_params=pltpu.CompilerParams(use_tc_tiling_on_sc=False),
        scratch_types=[
            pltpu.VMEM((PER_W,), jnp.int32),
            pltpu.VMEM((CHUNK, EM), jnp.float32),
            pltpu.SemaphoreType.DMA,
        ],
    )
    def gather_kernel(table_hbm, idx_hbm, out_hbm, idx_v, rows_v, sem):
        wid = lax.axis_index("s") * 2 + lax.axis_index("c")
        base = wid * PER_W
        pltpu.sync_copy(idx_hbm.at[pl.ds(base, PER_W)], idx_v)
        for c in range(NCH):
            pltpu.async_copy(
                table_hbm.at[idx_v.at[pl.ds(c * CHUNK, CHUNK)]], rows_v, sem
            ).wait()
            pltpu.sync_copy(rows_v, out_hbm.at[pl.ds(base + c * CHUNK, CHUNK)])

    return gather_kernel(table, idx_all)


def _score_body(gneg, gev, kd, w8, prm, inten_o, surv_o, ev8):
    n = pl.program_id(1)
    b0 = prm[0]
    b1 = prm[1]
    psi0 = prm[2]
    psi1 = prm[3]
    sp = lambda s, p: p * jnp.log1p(jnp.exp(s / p))
    dn = (((1,), (1,)), ((), ()))

    @pl.when(n == 0)
    def _():
        P = lax.dot_general(w8[...], gev[...], dn,
                            preferred_element_type=jnp.float32)
        ev8[...] = P
        k0 = kd[...] == 0
        sck = jnp.where(k0, P[0:1, :] + b0, P[1:2, :] + b1)
        psik = jnp.where(k0, psi0, psi1)
        inten_o[...] = psik * jnp.log1p(jnp.exp(sck / psik))

    Q = lax.dot_general(w8[...], gneg[...], dn,
                        preferred_element_type=jnp.float32)
    E = ev8[...]
    contrib = (sp(Q[4:5] + E[2:3] + b0, psi0)
               + sp(Q[5:6] + E[3:4] + b1, psi1)
               + sp(Q[2:3] + E[4:5] + b0, psi0)
               + sp(Q[3:4] + E[5:6] + b1, psi1)) * (1.0 / NNEG)

    @pl.when(n == 0)
    def _():
        surv_o[...] = contrib

    @pl.when(n > 0)
    def _():
        surv_o[...] += contrib


def _score_tc(g128, kd, w8, prm):
    out_shapes = (
        jax.ShapeDtypeStruct((1, B), jnp.float32),
        jax.ShapeDtypeStruct((1, B), jnp.float32),
    )
    return pl.pallas_call(
        _score_body,
        grid=(GRID, NNEG),
        in_specs=[
            pl.BlockSpec((BE, 2 * EM), lambda i, n: (n * GRID + i, 0)),
            pl.BlockSpec((BE, 2 * EM), lambda i, n: (NEG_BLKS + i, 0)),
            pl.BlockSpec((1, BE), lambda i, n: (0, i)),
            pl.BlockSpec((8, 2 * EM), lambda i, n: (0, 0)),
            pl.BlockSpec(memory_space=pltpu.SMEM),
        ],
        out_specs=(
            pl.BlockSpec((1, BE), lambda i, n: (0, i)),
            pl.BlockSpec((1, BE), lambda i, n: (0, i)),
        ),
        out_shape=out_shapes,
        scratch_shapes=[pltpu.VMEM((8, BE), jnp.float32)],
    )(g128, g128, kd, w8, prm)


def kernel(table, W0, b0, W1, b1, psi, events, negs):
    v1 = events[:, 0].astype(jnp.int32)
    v2 = events[:, 1].astype(jnp.int32)
    kd = events[:, 4].astype(jnp.int32)[None, :]

    neg_flat = jnp.transpose(negs.astype(jnp.int32), (1, 0, 2)).reshape(-1)
    ev_flat = jnp.stack([v1, v2], axis=1).reshape(-1)
    idx_all = jnp.concatenate([neg_flat, ev_flat])

    g = _gather_rows_sc(table, idx_all)
    g128 = g.reshape(R // 2, 2 * EM)

    w0a = W0[:EM, 0]
    w0b = W0[EM:, 0]
    w1a = W1[:EM, 0]
    w1b = W1[EM:, 0]
    z = jnp.zeros((EM,), jnp.float32)
    w8 = jnp.stack([
        jnp.concatenate([w0a, w0b]),
        jnp.concatenate([w1a, w1b]),
        jnp.concatenate([w0a, z]),
        jnp.concatenate([w1a, z]),
        jnp.concatenate([z, w0b]),
        jnp.concatenate([z, w1b]),
        jnp.zeros((2 * EM,), jnp.float32),
        jnp.zeros((2 * EM,), jnp.float32),
    ])
    prm = jnp.stack([b0[0], b1[0], psi[0, 0], psi[1, 0]])

    inten, surv = _score_tc(g128, kd, w8, prm)
    return inten, surv


# speedup vs baseline: 1.2843x; 1.0215x over previous
import functools

import jax
import jax.numpy as jnp
from jax import lax
from jax.experimental import pallas as pl
from jax.experimental.pallas import tpu as pltpu
from jax.experimental.pallas import tpu_sc as plsc

NSIZE = 1000000
EM = 64
B = 4096
NNEG = 20

NW = 32
R = B * (2 + 2 * NNEG)
RP = R // 2
PER_W = RP // NW
CHUNK = 672
NCH = PER_W // CHUNK

BE = 512
GRID = B // BE
NEG_BLKS = B * NNEG // BE


def _gather_rows_sc(table, idx_l, idx_r):
    mesh = plsc.VectorSubcoreMesh(core_axis_name="c", subcore_axis_name="s")

    @functools.partial(
        pl.kernel,
        out_type=jax.ShapeDtypeStruct((RP, 2 * EM), jnp.float32),
        mesh=mesh,
        compiler_params=pltpu.CompilerParams(use_tc_tiling_on_sc=False),
        scratch_types=[
            pltpu.VMEM((PER_W,), jnp.int32),
            pltpu.VMEM((PER_W,), jnp.int32),
            pltpu.VMEM((CHUNK, EM), jnp.float32),
            pltpu.VMEM((CHUNK, EM), jnp.float32),
            pltpu.SemaphoreType.DMA,
        ],
    )
    def gather_kernel(table_hbm, il_hbm, ir_hbm, out_hbm,
                      il_v, ir_v, rl_v, rr_v, sem):
        wid = lax.axis_index("s") * 2 + lax.axis_index("c")
        base = wid * PER_W
        pltpu.sync_copy(il_hbm.at[pl.ds(base, PER_W)], il_v)
        pltpu.sync_copy(ir_hbm.at[pl.ds(base, PER_W)], ir_v)
        for c in range(NCH):
            cl = pltpu.async_copy(
                table_hbm.at[il_v.at[pl.ds(c * CHUNK, CHUNK)]], rl_v, sem)
            cr = pltpu.async_copy(
                table_hbm.at[ir_v.at[pl.ds(c * CHUNK, CHUNK)]], rr_v, sem)
            cl.wait()
            cr.wait()
            row0 = base + c * CHUNK
            pltpu.sync_copy(rl_v, out_hbm.at[pl.ds(row0, CHUNK), pl.ds(0, EM)])
            pltpu.sync_copy(rr_v, out_hbm.at[pl.ds(row0, CHUNK), pl.ds(EM, EM)])

    return gather_kernel(table, idx_l, idx_r)


def _score_body(gneg, gev, kd, w8, prm, inten_o, surv_o, ev8):
    n = pl.program_id(1)
    b0 = prm[0]
    b1 = prm[1]
    psi0 = prm[2]
    psi1 = prm[3]
    sp = lambda s, p: p * jnp.log1p(jnp.exp(s / p))
    dn = (((1,), (1,)), ((), ()))

    @pl.when(n == 0)
    def _():
        P = lax.dot_general(w8[...], gev[...], dn,
                            preferred_element_type=jnp.float32)
        ev8[...] = P
        k0 = kd[...] == 0
        sck = jnp.where(k0, P[0:1, :] + b0, P[1:2, :] + b1)
        psik = jnp.where(k0, psi0, psi1)
        inten_o[...] = psik * jnp.log1p(jnp.exp(sck / psik))

    Q = lax.dot_general(w8[...], gneg[...], dn,
                        preferred_element_type=jnp.float32)
    E = ev8[...]
    contrib = (sp(Q[4:5] + E[2:3] + b0, psi0)
               + sp(Q[5:6] + E[3:4] + b1, psi1)
               + sp(Q[2:3] + E[4:5] + b0, psi0)
               + sp(Q[3:4] + E[5:6] + b1, psi1)) * (1.0 / NNEG)

    @pl.when(n == 0)
    def _():
        surv_o[...] = contrib

    @pl.when(n > 0)
    def _():
        surv_o[...] += contrib


def _score_tc(g128, kd, w8, prm):
    out_shapes = (
        jax.ShapeDtypeStruct((1, B), jnp.float32),
        jax.ShapeDtypeStruct((1, B), jnp.float32),
    )
    return pl.pallas_call(
        _score_body,
        grid=(GRID, NNEG),
        in_specs=[
            pl.BlockSpec((BE, 2 * EM), lambda i, n: (n * GRID + i, 0)),
            pl.BlockSpec((BE, 2 * EM), lambda i, n: (NEG_BLKS + i, 0)),
            pl.BlockSpec((1, BE), lambda i, n: (0, i)),
            pl.BlockSpec((8, 2 * EM), lambda i, n: (0, 0)),
            pl.BlockSpec(memory_space=pltpu.SMEM),
        ],
        out_specs=(
            pl.BlockSpec((1, BE), lambda i, n: (0, i)),
            pl.BlockSpec((1, BE), lambda i, n: (0, i)),
        ),
        out_shape=out_shapes,
        scratch_shapes=[pltpu.VMEM((8, BE), jnp.float32)],
    )(g128, g128, kd, w8, prm)


def kernel(table, W0, b0, W1, b1, psi, events, negs):
    v1 = events[:, 0].astype(jnp.int32)
    v2 = events[:, 1].astype(jnp.int32)
    kd = events[:, 4].astype(jnp.int32)[None, :]

    negT = jnp.transpose(negs.astype(jnp.int32), (1, 0, 2))
    idx_l = jnp.concatenate([negT[:, :, 0].reshape(-1), v1])
    idx_r = jnp.concatenate([negT[:, :, 1].reshape(-1), v2])

    g128 = _gather_rows_sc(table, idx_l, idx_r)

    w0a = W0[:EM, 0]
    w0b = W0[EM:, 0]
    w1a = W1[:EM, 0]
    w1b = W1[EM:, 0]
    z = jnp.zeros((EM,), jnp.float32)
    w8 = jnp.stack([
        jnp.concatenate([w0a, w0b]),
        jnp.concatenate([w1a, w1b]),
        jnp.concatenate([w0a, z]),
        jnp.concatenate([w1a, z]),
        jnp.concatenate([z, w0b]),
        jnp.concatenate([z, w1b]),
        jnp.zeros((2 * EM,), jnp.float32),
        jnp.zeros((2 * EM,), jnp.float32),
    ])
    prm = jnp.stack([b0[0], b1[0], psi[0, 0], psi[1, 0]])

    inten, surv = _score_tc(g128, kd, w8, prm)
    return inten, surv


# speedup vs baseline: 2.5819x; 2.0104x over previous
import functools

import jax
import jax.numpy as jnp
from jax import lax
from jax.experimental import pallas as pl
from jax.experimental.pallas import tpu as pltpu
from jax.experimental.pallas import tpu_sc as plsc

NSIZE = 1000000
EM = 64
B = 4096
NNEG = 20

NW = 32
RL = B * NNEG + B
PER_W = RL // NW
CHUNK = 672
NCH = PER_W // CHUNK

PS = 16384
PGRID = (NSIZE + PS - 1) // PS

BE = 512
GRID = B // BE
NEG_BLKS = B * NNEG // BE


def _project_body(tblT, w8, o0, o1, o2, o3):
    P = lax.dot_general(w8[...], tblT[...], (((1,), (0,)), ((), ())),
                        preferred_element_type=jnp.float32)
    o0[...] = P[0:1, :]
    o1[...] = P[1:2, :]
    o2[...] = P[2:3, :]
    o3[...] = P[3:4, :]


def _project_tc(tableT, w8):
    out_shapes = tuple(
        jax.ShapeDtypeStruct((1, NSIZE), jnp.float32) for _ in range(4))
    return pl.pallas_call(
        _project_body,
        grid=(PGRID,),
        in_specs=[
            pl.BlockSpec((EM, PS), lambda i: (0, i)),
            pl.BlockSpec((8, EM), lambda i: (0, 0)),
        ],
        out_specs=tuple(pl.BlockSpec((1, PS), lambda i: (0, i))
                        for _ in range(4)),
        out_shape=out_shapes,
    )(tableT, w8)


def _gather_proj_sc(p0, p1, p2, p3, idx_l, idx_r):
    mesh = plsc.VectorSubcoreMesh(core_axis_name="c", subcore_axis_name="s")

    @functools.partial(
        pl.kernel,
        out_type=tuple(
            jax.ShapeDtypeStruct((1, RL), jnp.float32) for _ in range(4)),
        mesh=mesh,
        compiler_params=pltpu.CompilerParams(use_tc_tiling_on_sc=False),
        scratch_types=[
            pltpu.VMEM((PER_W,), jnp.int32),
            pltpu.VMEM((PER_W,), jnp.int32),
            pltpu.VMEM((CHUNK,), jnp.float32),
            pltpu.VMEM((CHUNK,), jnp.float32),
            pltpu.VMEM((CHUNK,), jnp.float32),
            pltpu.VMEM((CHUNK,), jnp.float32),
            pltpu.SemaphoreType.DMA,
        ],
    )
    def gather_kernel(p0_h, p1_h, p2_h, p3_h, il_h, ir_h,
                      o0_h, o1_h, o2_h, o3_h,
                      il_v, ir_v, s0, s1, s2, s3, sem):
        wid = lax.axis_index("s") * 2 + lax.axis_index("c")
        base = wid * PER_W
        pltpu.sync_copy(il_h.at[pl.ds(base, PER_W)], il_v)
        pltpu.sync_copy(ir_h.at[pl.ds(base, PER_W)], ir_v)
        for c in range(NCH):
            il_c = il_v.at[pl.ds(c * CHUNK, CHUNK)]
            ir_c = ir_v.at[pl.ds(c * CHUNK, CHUNK)]
            cps = [
                pltpu.async_copy(p0_h.at[0].at[il_c], s0, sem),
                pltpu.async_copy(p1_h.at[0].at[il_c], s1, sem),
                pltpu.async_copy(p2_h.at[0].at[ir_c], s2, sem),
                pltpu.async_copy(p3_h.at[0].at[ir_c], s3, sem),
            ]
            for cp in cps:
                cp.wait()
            dst = pl.ds(base + c * CHUNK, CHUNK)
            pltpu.sync_copy(s0, o0_h.at[0, dst])
            pltpu.sync_copy(s1, o1_h.at[0, dst])
            pltpu.sync_copy(s2, o2_h.at[0, dst])
            pltpu.sync_copy(s3, o3_h.at[0, dst])

    return gather_kernel(p0, p1, p2, p3, idx_l, idx_r)


def _score_body(nA0, nA1, nB0, nB1, eA0, eA1, eB0, eB1, kd, prm,
                inten_o, surv_o):
    n = pl.program_id(1)
    b0 = prm[0]
    b1 = prm[1]
    psi0 = prm[2]
    psi1 = prm[3]
    sp = lambda s, p: p * jnp.log1p(jnp.exp(s / p))

    a0e = eA0[...]
    a1e = eA1[...]
    b0e = eB0[...]
    b1e = eB1[...]

    @pl.when(n == 0)
    def _():
        sc0 = a0e + b0e + b0
        sc1 = a1e + b1e + b1
        k0 = kd[...] == 0
        sck = jnp.where(k0, sc0, sc1)
        psik = jnp.where(k0, psi0, psi1)
        inten_o[...] = psik * jnp.log1p(jnp.exp(sck / psik))

    contrib = (sp(a0e + nB0[...] + b0, psi0)
               + sp(a1e + nB1[...] + b1, psi1)
               + sp(nA0[...] + b0e + b0, psi0)
               + sp(nA1[...] + b1e + b1, psi1)) * (1.0 / NNEG)

    @pl.when(n == 0)
    def _():
        surv_o[...] = contrib

    @pl.when(n > 0)
    def _():
        surv_o[...] += contrib


def _score_tc(a0, a1, bb0, bb1, kd, prm):
    neg_spec = pl.BlockSpec((1, BE), lambda i, n: (0, n * GRID + i))
    ev_spec = pl.BlockSpec((1, BE), lambda i, n: (0, NEG_BLKS + i))
    out_shapes = (
        jax.ShapeDtypeStruct((1, B), jnp.float32),
        jax.ShapeDtypeStruct((1, B), jnp.float32),
    )
    return pl.pallas_call(
        _score_body,
        grid=(GRID, NNEG),
        in_specs=[neg_spec, neg_spec, neg_spec, neg_spec,
                  ev_spec, ev_spec, ev_spec, ev_spec,
                  pl.BlockSpec((1, BE), lambda i, n: (0, i)),
                  pl.BlockSpec(memory_space=pltpu.SMEM)],
        out_specs=(
            pl.BlockSpec((1, BE), lambda i, n: (0, i)),
            pl.BlockSpec((1, BE), lambda i, n: (0, i)),
        ),
        out_shape=out_shapes,
    )(a0, a1, bb0, bb1, a0, a1, bb0, bb1, kd, prm)


def kernel(table, W0, b0, W1, b1, psi, events, negs):
    v1 = events[:, 0].astype(jnp.int32)
    v2 = events[:, 1].astype(jnp.int32)
    kd = events[:, 4].astype(jnp.int32)[None, :]

    negT = jnp.transpose(negs.astype(jnp.int32), (1, 0, 2))
    idx_l = jnp.concatenate([negT[:, :, 0].reshape(-1), v1])
    idx_r = jnp.concatenate([negT[:, :, 1].reshape(-1), v2])

    tableT = jnp.swapaxes(table, 0, 1)
    w0a = W0[:EM, 0]
    w0b = W0[EM:, 0]
    w1a = W1[:EM, 0]
    w1b = W1[EM:, 0]
    w8 = jnp.stack([w0a, w1a, w0b, w1b] + [jnp.zeros((EM,), jnp.float32)] * 4)

    p0, p1, p2, p3 = _project_tc(tableT, w8)
    a0, a1, bb0, bb1 = _gather_proj_sc(p0, p1, p2, p3, idx_l, idx_r)

    prm = jnp.stack([b0[0], b1[0], psi[0, 0], psi[1, 0]])
    inten, surv = _score_tc(a0, a1, bb0, bb1, kd, prm)
    return inten, surv


# speedup vs baseline: 4.7958x; 1.8574x over previous
import functools

import jax
import jax.numpy as jnp
from jax import lax
from jax.experimental import pallas as pl
from jax.experimental.pallas import tpu as pltpu
from jax.experimental.pallas import tpu_sc as plsc

NSIZE = 1000000
EM = 64
B = 4096
NNEG = 20

NW = 32
RL = B * NNEG + B
PER_W = RL // NW
CHUNK = 672
NCH = PER_W // CHUNK

PS = 16384
PGRID = (NSIZE + PS - 1) // PS

BE = 512
GRID = B // BE
NEG_BLKS = B * NNEG // BE


def _project_body(tblT, w8, o0, o1, o2, o3):
    P = lax.dot_general(w8[...], tblT[...], (((1,), (0,)), ((), ())),
                        preferred_element_type=jnp.float32)
    o0[...] = P[0, :]
    o1[...] = P[1, :]
    o2[...] = P[2, :]
    o3[...] = P[3, :]


def _project_tc(tableT, w8):
    out_shapes = tuple(
        jax.ShapeDtypeStruct((NSIZE,), jnp.float32) for _ in range(4))
    return pl.pallas_call(
        _project_body,
        grid=(PGRID,),
        in_specs=[
            pl.BlockSpec((EM, PS), lambda i: (0, i)),
            pl.BlockSpec((8, EM), lambda i: (0, 0)),
        ],
        out_specs=tuple(pl.BlockSpec((PS,), lambda i: (i,))
                        for _ in range(4)),
        out_shape=out_shapes,
    )(tableT, w8)


def _gather_proj_sc(p0, p1, p2, p3, idx_l, idx_r):
    mesh = plsc.VectorSubcoreMesh(core_axis_name="c", subcore_axis_name="s")

    @functools.partial(
        pl.kernel,
        out_type=tuple(
            jax.ShapeDtypeStruct((1, RL), jnp.float32) for _ in range(4)),
        mesh=mesh,
        compiler_params=pltpu.CompilerParams(use_tc_tiling_on_sc=False),
        scratch_types=[
            pltpu.VMEM((PER_W,), jnp.int32),
            pltpu.VMEM((PER_W,), jnp.int32),
            pltpu.VMEM((CHUNK,), jnp.float32),
            pltpu.VMEM((CHUNK,), jnp.float32),
            pltpu.VMEM((CHUNK,), jnp.float32),
            pltpu.VMEM((CHUNK,), jnp.float32),
            pltpu.SemaphoreType.DMA,
        ],
    )
    def gather_kernel(p0_h, p1_h, p2_h, p3_h, il_h, ir_h,
                      o0_h, o1_h, o2_h, o3_h,
                      il_v, ir_v, s0, s1, s2, s3, sem):
        wid = lax.axis_index("s") * 2 + lax.axis_index("c")
        base = wid * PER_W
        pltpu.sync_copy(il_h.at[pl.ds(base, PER_W)], il_v)
        pltpu.sync_copy(ir_h.at[pl.ds(base, PER_W)], ir_v)
        for c in range(NCH):
            il_c = il_v.at[pl.ds(c * CHUNK, CHUNK)]
            ir_c = ir_v.at[pl.ds(c * CHUNK, CHUNK)]
            cps = [
                pltpu.async_copy(p0_h.at[il_c], s0, sem),
                pltpu.async_copy(p1_h.at[il_c], s1, sem),
                pltpu.async_copy(p2_h.at[ir_c], s2, sem),
                pltpu.async_copy(p3_h.at[ir_c], s3, sem),
            ]
            for cp in cps:
                cp.wait()
            dst = pl.ds(base + c * CHUNK, CHUNK)
            pltpu.sync_copy(s0, o0_h.at[0, dst])
            pltpu.sync_copy(s1, o1_h.at[0, dst])
            pltpu.sync_copy(s2, o2_h.at[0, dst])
            pltpu.sync_copy(s3, o3_h.at[0, dst])

    return gather_kernel(p0, p1, p2, p3, idx_l, idx_r)


def _score_body(nA0, nA1, nB0, nB1, eA0, eA1, eB0, eB1, kd, prm,
                inten_o, surv_o):
    n = pl.program_id(1)
    b0 = prm[0]
    b1 = prm[1]
    psi0 = prm[2]
    psi1 = prm[3]
    sp = lambda s, p: p * jnp.log1p(jnp.exp(s / p))

    a0e = eA0[...]
    a1e = eA1[...]
    b0e = eB0[...]
    b1e = eB1[...]

    @pl.when(n == 0)
    def _():
        sc0 = a0e + b0e + b0
        sc1 = a1e + b1e + b1
        k0 = kd[...] == 0
        sck = jnp.where(k0, sc0, sc1)
        psik = jnp.where(k0, psi0, psi1)
        inten_o[...] = psik * jnp.log1p(jnp.exp(sck / psik))

    contrib = (sp(a0e + nB0[...] + b0, psi0)
               + sp(a1e + nB1[...] + b1, psi1)
               + sp(nA0[...] + b0e + b0, psi0)
               + sp(nA1[...] + b1e + b1, psi1)) * (1.0 / NNEG)

    @pl.when(n == 0)
    def _():
        surv_o[...] = contrib

    @pl.when(n > 0)
    def _():
        surv_o[...] += contrib


def _score_tc(a0, a1, bb0, bb1, kd, prm):
    neg_spec = pl.BlockSpec((1, BE), lambda i, n: (0, n * GRID + i))
    ev_spec = pl.BlockSpec((1, BE), lambda i, n: (0, NEG_BLKS + i))
    out_shapes = (
        jax.ShapeDtypeStruct((1, B), jnp.float32),
        jax.ShapeDtypeStruct((1, B), jnp.float32),
    )
    return pl.pallas_call(
        _score_body,
        grid=(GRID, NNEG),
        in_specs=[neg_spec, neg_spec, neg_spec, neg_spec,
                  ev_spec, ev_spec, ev_spec, ev_spec,
                  pl.BlockSpec((1, BE), lambda i, n: (0, i)),
                  pl.BlockSpec(memory_space=pltpu.SMEM)],
        out_specs=(
            pl.BlockSpec((1, BE), lambda i, n: (0, i)),
            pl.BlockSpec((1, BE), lambda i, n: (0, i)),
        ),
        out_shape=out_shapes,
    )(a0, a1, bb0, bb1, a0, a1, bb0, bb1, kd, prm)


def kernel(table, W0, b0, W1, b1, psi, events, negs):
    v1 = events[:, 0].astype(jnp.int32)
    v2 = events[:, 1].astype(jnp.int32)
    kd = events[:, 4].astype(jnp.int32)[None, :]

    negT = jnp.transpose(negs.astype(jnp.int32), (1, 0, 2))
    idx_l = jnp.concatenate([negT[:, :, 0].reshape(-1), v1])
    idx_r = jnp.concatenate([negT[:, :, 1].reshape(-1), v2])

    tableT = jnp.swapaxes(table, 0, 1)
    w0a = W0[:EM, 0]
    w0b = W0[EM:, 0]
    w1a = W1[:EM, 0]
    w1b = W1[EM:, 0]
    w8 = jnp.stack([w0a, w1a, w0b, w1b] + [jnp.zeros((EM,), jnp.float32)] * 4)

    p0, p1, p2, p3 = _project_tc(tableT, w8)
    a0, a1, bb0, bb1 = _gather_proj_sc(p0, p1, p2, p3, idx_l, idx_r)

    prm = jnp.stack([b0[0], b1[0], psi[0, 0], psi[1, 0]])
    inten, surv = _score_tc(a0, a1, bb0, bb1, kd, prm)
    return inten, surv


# speedup vs baseline: 6.7285x; 1.4030x over previous
import functools

import jax
import jax.numpy as jnp
from jax import lax
from jax.experimental import pallas as pl
from jax.experimental.pallas import tpu as pltpu
from jax.experimental.pallas import tpu_sc as plsc

NSIZE = 1000000
EM = 64
B = 4096
NNEG = 20

NW = 32
RL = B * NNEG + B
PER_W = RL // NW
CHUNK = 672
NCH = PER_W // CHUNK

PS = 16384
PGRID = (NSIZE + PS - 1) // PS

BE = B
NEG_BLKS = B * NNEG // BE


def _project_body(tblT, w8, o0, o1, o2, o3):
    P = lax.dot_general(w8[...], tblT[...], (((1,), (0,)), ((), ())),
                        preferred_element_type=jnp.float32)
    o0[...] = P[0, :]
    o1[...] = P[1, :]
    o2[...] = P[2, :]
    o3[...] = P[3, :]


def _project_tc(tableT, w8):
    out_shapes = tuple(
        jax.ShapeDtypeStruct((NSIZE,), jnp.float32) for _ in range(4))
    return pl.pallas_call(
        _project_body,
        grid=(PGRID,),
        in_specs=[
            pl.BlockSpec((EM, PS), lambda i: (0, i)),
            pl.BlockSpec((8, EM), lambda i: (0, 0)),
        ],
        out_specs=tuple(pl.BlockSpec((PS,), lambda i: (i,))
                        for _ in range(4)),
        out_shape=out_shapes,
    )(tableT, w8)


def _gather_proj_sc(p0, p1, p2, p3, idx_l, idx_r):
    mesh = plsc.VectorSubcoreMesh(core_axis_name="c", subcore_axis_name="s")

    @functools.partial(
        pl.kernel,
        out_type=tuple(
            jax.ShapeDtypeStruct((1, RL), jnp.float32) for _ in range(4)),
        mesh=mesh,
        compiler_params=pltpu.CompilerParams(use_tc_tiling_on_sc=False),
        scratch_types=[
            pltpu.VMEM((PER_W,), jnp.int32),
            pltpu.VMEM((PER_W,), jnp.int32),
            pltpu.VMEM((CHUNK,), jnp.float32),
            pltpu.VMEM((CHUNK,), jnp.float32),
            pltpu.VMEM((CHUNK,), jnp.float32),
            pltpu.VMEM((CHUNK,), jnp.float32),
            pltpu.SemaphoreType.DMA,
        ],
    )
    def gather_kernel(p0_h, p1_h, p2_h, p3_h, il_h, ir_h,
                      o0_h, o1_h, o2_h, o3_h,
                      il_v, ir_v, s0, s1, s2, s3, sem):
        wid = lax.axis_index("s") * 2 + lax.axis_index("c")
        base = wid * PER_W
        pltpu.sync_copy(il_h.at[pl.ds(base, PER_W)], il_v)
        pltpu.sync_copy(ir_h.at[pl.ds(base, PER_W)], ir_v)
        for c in range(NCH):
            il_c = il_v.at[pl.ds(c * CHUNK, CHUNK)]
            ir_c = ir_v.at[pl.ds(c * CHUNK, CHUNK)]
            cps = [
                pltpu.async_copy(p0_h.at[il_c], s0, sem),
                pltpu.async_copy(p1_h.at[il_c], s1, sem),
                pltpu.async_copy(p2_h.at[ir_c], s2, sem),
                pltpu.async_copy(p3_h.at[ir_c], s3, sem),
            ]
            for cp in cps:
                cp.wait()
            dst = pl.ds(base + c * CHUNK, CHUNK)
            pltpu.sync_copy(s0, o0_h.at[0, dst])
            pltpu.sync_copy(s1, o1_h.at[0, dst])
            pltpu.sync_copy(s2, o2_h.at[0, dst])
            pltpu.sync_copy(s3, o3_h.at[0, dst])

    return gather_kernel(p0, p1, p2, p3, idx_l, idx_r)


def _score_body(nA0, nA1, nB0, nB1, eA0, eA1, eB0, eB1, kd, prm,
                inten_o, surv_o):
    n = pl.program_id(0)
    b0 = prm[0]
    b1 = prm[1]
    psi0 = prm[2]
    psi1 = prm[3]
    sp = lambda s, p: p * jnp.log1p(jnp.exp(s / p))

    a0e = eA0[...]
    a1e = eA1[...]
    b0e = eB0[...]
    b1e = eB1[...]

    @pl.when(n == 0)
    def _():
        sc0 = a0e + b0e + b0
        sc1 = a1e + b1e + b1
        k0 = kd[...] == 0
        sck = jnp.where(k0, sc0, sc1)
        psik = jnp.where(k0, psi0, psi1)
        inten_o[...] = psik * jnp.log1p(jnp.exp(sck / psik))

    contrib = (sp(a0e + nB0[...] + b0, psi0)
               + sp(a1e + nB1[...] + b1, psi1)
               + sp(nA0[...] + b0e + b0, psi0)
               + sp(nA1[...] + b1e + b1, psi1)) * (1.0 / NNEG)

    @pl.when(n == 0)
    def _():
        surv_o[...] = contrib

    @pl.when(n > 0)
    def _():
        surv_o[...] += contrib


def _score_tc(a0, a1, bb0, bb1, kd, prm):
    neg_spec = pl.BlockSpec((1, BE), lambda n: (0, n))
    ev_spec = pl.BlockSpec((1, BE), lambda n: (0, NEG_BLKS))
    out_spec = pl.BlockSpec((1, BE), lambda n: (0, 0))
    out_shapes = (
        jax.ShapeDtypeStruct((1, B), jnp.float32),
        jax.ShapeDtypeStruct((1, B), jnp.float32),
    )
    return pl.pallas_call(
        _score_body,
        grid=(NNEG,),
        in_specs=[neg_spec, neg_spec, neg_spec, neg_spec,
                  ev_spec, ev_spec, ev_spec, ev_spec,
                  out_spec,
                  pl.BlockSpec(memory_space=pltpu.SMEM)],
        out_specs=(out_spec, out_spec),
        out_shape=out_shapes,
    )(a0, a1, bb0, bb1, a0, a1, bb0, bb1, kd, prm)


def kernel(table, W0, b0, W1, b1, psi, events, negs):
    v1 = events[:, 0].astype(jnp.int32)
    v2 = events[:, 1].astype(jnp.int32)
    kd = events[:, 4].astype(jnp.int32)[None, :]

    negT = jnp.transpose(negs.astype(jnp.int32), (1, 0, 2))
    idx_l = jnp.concatenate([negT[:, :, 0].reshape(-1), v1])
    idx_r = jnp.concatenate([negT[:, :, 1].reshape(-1), v2])

    tableT = jnp.swapaxes(table, 0, 1)
    w0a = W0[:EM, 0]
    w0b = W0[EM:, 0]
    w1a = W1[:EM, 0]
    w1b = W1[EM:, 0]
    w8 = jnp.stack([w0a, w1a, w0b, w1b] + [jnp.zeros((EM,), jnp.float32)] * 4)

    p0, p1, p2, p3 = _project_tc(tableT, w8)
    a0, a1, bb0, bb1 = _gather_proj_sc(p0, p1, p2, p3, idx_l, idx_r)

    prm = jnp.stack([b0[0], b1[0], psi[0, 0], psi[1, 0]])
    inten, surv = _score_tc(a0, a1, bb0, bb1, kd, prm)
    return inten, surv


# speedup vs baseline: 7.1702x; 1.0656x over previous
import functools

import jax
import jax.numpy as jnp
from jax import lax
from jax.experimental import pallas as pl
from jax.experimental.pallas import tpu as pltpu
from jax.experimental.pallas import tpu_sc as plsc

NSIZE = 1000000
EM = 64
B = 4096
NNEG = 20

NW = 32
RL = B * NNEG + B
PER_W = RL // NW
CHUNK = 672
NCH = PER_W // CHUNK

PS = 32768
PGRID = (NSIZE + PS - 1) // PS

BE = B
NEG_BLKS = B * NNEG // BE


def _project_body(tblT, w8, o0, o1, o2, o3):
    P = lax.dot_general(w8[...], tblT[...], (((1,), (0,)), ((), ())),
                        preferred_element_type=jnp.float32)
    o0[...] = P[0, :]
    o1[...] = P[1, :]
    o2[...] = P[2, :]
    o3[...] = P[3, :]


def _project_tc(tableT, w8):
    out_shapes = tuple(
        jax.ShapeDtypeStruct((NSIZE,), jnp.float32) for _ in range(4))
    return pl.pallas_call(
        _project_body,
        grid=(PGRID,),
        in_specs=[
            pl.BlockSpec((EM, PS), lambda i: (0, i)),
            pl.BlockSpec((8, EM), lambda i: (0, 0)),
        ],
        out_specs=tuple(pl.BlockSpec((PS,), lambda i: (i,))
                        for _ in range(4)),
        out_shape=out_shapes,
    )(tableT, w8)


def _gather_proj_sc(p0, p1, p2, p3, idx_l, idx_r):
    mesh = plsc.VectorSubcoreMesh(core_axis_name="c", subcore_axis_name="s")

    @functools.partial(
        pl.kernel,
        out_type=tuple(
            jax.ShapeDtypeStruct((1, RL), jnp.float32) for _ in range(4)),
        mesh=mesh,
        compiler_params=pltpu.CompilerParams(use_tc_tiling_on_sc=False),
        scratch_types=[
            pltpu.VMEM((PER_W,), jnp.int32),
            pltpu.VMEM((PER_W,), jnp.int32),
            pltpu.VMEM((CHUNK,), jnp.float32),
            pltpu.VMEM((CHUNK,), jnp.float32),
            pltpu.VMEM((CHUNK,), jnp.float32),
            pltpu.VMEM((CHUNK,), jnp.float32),
            pltpu.SemaphoreType.DMA,
        ],
    )
    def gather_kernel(p0_h, p1_h, p2_h, p3_h, il_h, ir_h,
                      o0_h, o1_h, o2_h, o3_h,
                      il_v, ir_v, s0, s1, s2, s3, sem):
        wid = lax.axis_index("s") * 2 + lax.axis_index("c")
        base = wid * PER_W
        pltpu.sync_copy(il_h.at[pl.ds(base, PER_W)], il_v)
        pltpu.sync_copy(ir_h.at[pl.ds(base, PER_W)], ir_v)
        for c in range(NCH):
            il_c = il_v.at[pl.ds(c * CHUNK, CHUNK)]
            ir_c = ir_v.at[pl.ds(c * CHUNK, CHUNK)]
            cps = [
                pltpu.async_copy(p0_h.at[il_c], s0, sem),
                pltpu.async_copy(p1_h.at[il_c], s1, sem),
                pltpu.async_copy(p2_h.at[ir_c], s2, sem),
                pltpu.async_copy(p3_h.at[ir_c], s3, sem),
            ]
            for cp in cps:
                cp.wait()
            dst = pl.ds(base + c * CHUNK, CHUNK)
            pltpu.sync_copy(s0, o0_h.at[0, dst])
            pltpu.sync_copy(s1, o1_h.at[0, dst])
            pltpu.sync_copy(s2, o2_h.at[0, dst])
            pltpu.sync_copy(s3, o3_h.at[0, dst])

    return gather_kernel(p0, p1, p2, p3, idx_l, idx_r)


def _score_body(nA0, nA1, nB0, nB1, eA0, eA1, eB0, eB1, kd, prm,
                inten_o, surv_o):
    n = pl.program_id(0)
    b0 = prm[0]
    b1 = prm[1]
    psi0 = prm[2]
    psi1 = prm[3]
    sp = lambda s, p: p * jnp.log1p(jnp.exp(s / p))

    a0e = eA0[...]
    a1e = eA1[...]
    b0e = eB0[...]
    b1e = eB1[...]

    @pl.when(n == 0)
    def _():
        sc0 = a0e + b0e + b0
        sc1 = a1e + b1e + b1
        k0 = kd[...] == 0
        sck = jnp.where(k0, sc0, sc1)
        psik = jnp.where(k0, psi0, psi1)
        inten_o[...] = psik * jnp.log1p(jnp.exp(sck / psik))

    contrib = (sp(a0e + nB0[...] + b0, psi0)
               + sp(a1e + nB1[...] + b1, psi1)
               + sp(nA0[...] + b0e + b0, psi0)
               + sp(nA1[...] + b1e + b1, psi1)) * (1.0 / NNEG)

    @pl.when(n == 0)
    def _():
        surv_o[...] = contrib

    @pl.when(n > 0)
    def _():
        surv_o[...] += contrib


def _score_tc(a0, a1, bb0, bb1, kd, prm):
    neg_spec = pl.BlockSpec((1, BE), lambda n: (0, n))
    ev_spec = pl.BlockSpec((1, BE), lambda n: (0, NEG_BLKS))
    out_spec = pl.BlockSpec((1, BE), lambda n: (0, 0))
    out_shapes = (
        jax.ShapeDtypeStruct((1, B), jnp.float32),
        jax.ShapeDtypeStruct((1, B), jnp.float32),
    )
    return pl.pallas_call(
        _score_body,
        grid=(NNEG,),
        in_specs=[neg_spec, neg_spec, neg_spec, neg_spec,
                  ev_spec, ev_spec, ev_spec, ev_spec,
                  out_spec,
                  pl.BlockSpec(memory_space=pltpu.SMEM)],
        out_specs=(out_spec, out_spec),
        out_shape=out_shapes,
    )(a0, a1, bb0, bb1, a0, a1, bb0, bb1, kd, prm)


def kernel(table, W0, b0, W1, b1, psi, events, negs):
    v1 = events[:, 0].astype(jnp.int32)
    v2 = events[:, 1].astype(jnp.int32)
    kd = events[:, 4].astype(jnp.int32)[None, :]

    negT = jnp.transpose(negs.astype(jnp.int32), (1, 0, 2))
    idx_l = jnp.concatenate([negT[:, :, 0].reshape(-1), v1])
    idx_r = jnp.concatenate([negT[:, :, 1].reshape(-1), v2])

    tableT = jnp.swapaxes(table, 0, 1)
    w0a = W0[:EM, 0]
    w0b = W0[EM:, 0]
    w1a = W1[:EM, 0]
    w1b = W1[EM:, 0]
    w8 = jnp.stack([w0a, w1a, w0b, w1b] + [jnp.zeros((EM,), jnp.float32)] * 4)

    p0, p1, p2, p3 = _project_tc(tableT, w8)
    a0, a1, bb0, bb1 = _gather_proj_sc(p0, p1, p2, p3, idx_l, idx_r)

    prm = jnp.stack([b0[0], b1[0], psi[0, 0], psi[1, 0]])
    inten, surv = _score_tc(a0, a1, bb0, bb1, kd, prm)
    return inten, surv
